# baseline (speedup 1.0000x reference)
import functools

import jax
import jax.numpy as jnp
from jax import lax
from jax.experimental import pallas as pl
from jax.experimental.pallas import tpu as pltpu

_DEPTH = 3
_TILE = 4096
_CHUNK = 512
_VMEM_LIMIT = 64 * 1024 * 1024


def _gather_h(src_col, h, tile_e):
    n = h.shape[0]
    gsel = (src_col == lax.broadcasted_iota(jnp.int32, (tile_e, n), 1)
            ).astype(jnp.float32)
    return jnp.dot(gsel, h, preferred_element_type=jnp.float32)


def _scatter_sum(tgt_row, msg, n):
    tile_e = msg.shape[0]
    ssel = (tgt_row == lax.broadcasted_iota(jnp.int32, (n, tile_e), 0)
            ).astype(msg.dtype)
    contrib = jnp.dot(ssel, msg, preferred_element_type=jnp.float32)
    deg_part = jnp.sum(ssel.astype(jnp.float32), axis=1, keepdims=True)
    return contrib, deg_part


def _seg_reduce_mat(chunk, width, lo, dtype):
    jj = lax.broadcasted_iota(jnp.int32, (chunk, width), 0)
    vv = lax.broadcasted_iota(jnp.int32, (chunk, width), 1)
    return ((jj // width + lo // width) == vv).astype(dtype)


def _msg0_kernel(x_ref, fc1w_ref, fc1b_ref,
                 ea_ref, srcc_ref, tgtr_ref,
                 k1w_ref, k1b_ref, k2w_ref, k2b_ref, k3w_ref, k3b_ref,
                 part_ref, hout_ref, deg_ref, ewc_ref, h_vmem,
                 *, width, chunk):
    t = pl.program_id(0)
    tile_e = ea_ref.shape[0]
    n = h_vmem.shape[0]
    ww = k3w_ref.shape[1]

    @pl.when(t == 0)
    def _():
        h = (jnp.dot(x_ref[...], fc1w_ref[...],
                     preferred_element_type=jnp.float32) + fc1b_ref[...])
        h_vmem[...] = h
        hout_ref[...] = h
        part_ref[...] = jnp.zeros(part_ref.shape, jnp.float32)
        deg_ref[...] = jnp.zeros(deg_ref.shape, jnp.float32)

    eh = jnp.maximum(jnp.dot(ea_ref[...], k1w_ref[...],
                             preferred_element_type=jnp.float32)
                     + k1b_ref[...], 0.0)
    eh = jnp.maximum(jnp.dot(eh, k2w_ref[...],
                             preferred_element_type=jnp.float32)
                     + k2b_ref[...], 0.0)

    hs = _gather_h(srcc_ref[0], h_vmem[...], tile_e)
    hexp = pltpu.repeat(hs, chunk // width, axis=1)
    msg = jnp.zeros((tile_e, width), jnp.float32)
    for lo in range(0, ww, chunk):
        ew_c = (jnp.dot(eh, k3w_ref[:, lo:lo + chunk],
                        preferred_element_type=jnp.float32)
                + k3b_ref[:, lo:lo + chunk])
        ewc_ref[:, lo:lo + chunk] = ew_c.astype(jnp.bfloat16)
        prod = hexp * ew_c
        s2 = _seg_reduce_mat(chunk, width, lo, jnp.float32)
        msg = msg + jnp.dot(prod, s2, preferred_element_type=jnp.float32)

    contrib, deg_part = _scatter_sum(tgtr_ref[0], msg, n)
    part_ref[...] += contrib
    deg_ref[...] += deg_part


def _msgs_kernel(hprev_ref, pprev_ref, degin_ref, root_ref, cb_ref,
                 ewc_ref, srcc_ref, tgtr_ref,
                 part_ref, hout_ref, h_vmem, *, width, chunk):
    t = pl.program_id(0)
    tile_e = ewc_ref.shape[0]
    n = h_vmem.shape[0]
    ww = ewc_ref.shape[1]

    @pl.when(t == 0)
    def _():
        invd = 1.0 / jnp.maximum(degin_ref[...], 1.0)
        h = jnp.maximum(
            pprev_ref[...] * invd
            + jnp.dot(hprev_ref[...], root_ref[...],
                      preferred_element_type=jnp.float32)
            + cb_ref[...], 0.0)
        h_vmem[...] = h
        hout_ref[...] = h
        part_ref[...] = jnp.zeros(part_ref.shape, jnp.float32)

    hs = _gather_h(srcc_ref[0], h_vmem[...], tile_e)
    hexp = pltpu.repeat(hs.astype(jnp.bfloat16), chunk // width, axis=1)
    msg = jnp.zeros((tile_e, width), jnp.float32)
    for lo in range(0, ww, chunk):
        prod = hexp * ewc_ref[:, lo:lo + chunk]
        s2 = _seg_reduce_mat(chunk, width, lo, jnp.bfloat16)
        msg = msg + jnp.dot(prod, s2, preferred_element_type=jnp.float32)

    contrib, _ = _scatter_sum(tgtr_ref[0], msg, n)
    part_ref[...] += contrib


def _final_kernel(hprev_ref, pprev_ref, degin_ref, root_ref, cb_ref,
                  fc2w_ref, fc2b_ref, out_ref):
    invd = 1.0 / jnp.maximum(degin_ref[...], 1.0)
    h = jnp.maximum(
        pprev_ref[...] * invd
        + jnp.dot(hprev_ref[...], root_ref[...],
                  preferred_element_type=jnp.float32)
        + cb_ref[...], 0.0)
    out_ref[...] = (jnp.dot(h, fc2w_ref[...],
                            preferred_element_type=jnp.float32) + fc2b_ref[...])


def kernel(x, edge_index, edge_attr, fc1_w, fc1_b, k1_w, k1_b, k2_w, k2_b,
           k3_w, k3_b, root, conv_bias, fc2_w, fc2_b):
    n, _ = x.shape
    e_real, ker_in = edge_attr.shape
    width = root.shape[0]
    ww = width * width
    out_w = fc2_w.shape[1]
    tile = _TILE
    chunk = min(_CHUNK, ww)

    e_pad = ((e_real + tile - 1) // tile) * tile
    nt = e_pad // tile

    src = edge_index[0]
    tgt = edge_index[1]

    ker_in_pad = ((ker_in + 7) // 8) * 8
    ea = jnp.pad(edge_attr.astype(jnp.float32),
                 ((0, e_pad - e_real), (0, ker_in_pad - ker_in)))
    k1w = jnp.pad(k1_w, ((0, ker_in_pad - ker_in), (0, 0)))
    srcc = jnp.pad(src, (0, e_pad - e_real)).reshape(nt, tile, 1)
    tgtr = jnp.pad(tgt, (0, e_pad - e_real),
                   constant_values=n).reshape(nt, 1, tile)

    k3wp = k3_w.reshape(-1, width, width).transpose(0, 2, 1).reshape(-1, ww)
    k3bp = k3_b.reshape(1, width, width).transpose(0, 2, 1).reshape(1, ww)

    grid = (nt,)
    c2 = lambda t: (0, 0)
    ea_spec = pl.BlockSpec((tile, ker_in_pad), lambda t: (t, 0))
    ew_spec = pl.BlockSpec((tile, ww), lambda t: (t, 0))
    src_spec = pl.BlockSpec((1, tile, 1), lambda t: (t, 0, 0))
    tgt_spec = pl.BlockSpec((1, 1, tile), lambda t: (t, 0, 0))
    nw_spec = pl.BlockSpec((n, width), c2)
    nd_spec = pl.BlockSpec((n, 1), c2)
    cparams = pltpu.CompilerParams(
        dimension_semantics=("arbitrary",),
        vmem_limit_bytes=_VMEM_LIMIT)
    scratch = [pltpu.VMEM((n, width), jnp.float32)]

    p, hcur, deg, ewc = pl.pallas_call(
        functools.partial(_msg0_kernel, width=width, chunk=chunk),
        grid=grid,
        in_specs=[pl.BlockSpec(x.shape, c2), pl.BlockSpec(fc1_w.shape, c2),
                  pl.BlockSpec(fc1_b.shape, c2),
                  ea_spec, src_spec, tgt_spec,
                  pl.BlockSpec(k1w.shape, c2), pl.BlockSpec(k1_b.shape, c2),
                  pl.BlockSpec(k2_w.shape, c2), pl.BlockSpec(k2_b.shape, c2),
                  pl.BlockSpec(k3wp.shape, c2), pl.BlockSpec(k3bp.shape, c2)],
        out_specs=[nw_spec, nw_spec, nd_spec, ew_spec],
        out_shape=[jax.ShapeDtypeStruct((n, width), jnp.float32),
                   jax.ShapeDtypeStruct((n, width), jnp.float32),
                   jax.ShapeDtypeStruct((n, 1), jnp.float32),
                   jax.ShapeDtypeStruct((e_pad, ww), jnp.bfloat16)],
        scratch_shapes=scratch,
        compiler_params=cparams,
    )(x.astype(jnp.float32), fc1_w, fc1_b, ea, srcc, tgtr,
      k1w, k1_b, k2_w, k2_b, k3wp, k3bp)

    for _ in range(_DEPTH - 1):
        p, hcur = pl.pallas_call(
            functools.partial(_msgs_kernel, width=width, chunk=chunk),
            grid=grid,
            in_specs=[nw_spec, nw_spec, nd_spec,
                      pl.BlockSpec(root.shape, c2),
                      pl.BlockSpec(conv_bias.shape, c2),
                      ew_spec, src_spec, tgt_spec],
            out_specs=[nw_spec, nw_spec],
            out_shape=[jax.ShapeDtypeStruct((n, width), jnp.float32),
                       jax.ShapeDtypeStruct((n, width), jnp.float32)],
            scratch_shapes=scratch,
            compiler_params=cparams,
        )(hcur, p, deg, root, conv_bias, ewc, srcc, tgtr)

    return pl.pallas_call(
        _final_kernel,
        out_shape=jax.ShapeDtypeStruct((n, out_w), jnp.float32),
    )(hcur, p, deg, root, conv_bias, fc2_w, fc2_b)


# speedup vs baseline: 1.8016x; 1.8016x over previous
import functools

import jax
import jax.numpy as jnp
from jax import lax
from jax.experimental import pallas as pl
from jax.experimental.pallas import tpu as pltpu

_DEPTH = 3
_TILE = 4096
_CHUNK = 512
_VMEM_LIMIT = 64 * 1024 * 1024


def _gather_ht(src_row, ht, tile_e):
    n = ht.shape[1]
    gselt = (src_row == lax.broadcasted_iota(jnp.int32, (n, tile_e), 0)
             ).astype(jnp.float32)
    return jnp.dot(ht, gselt, preferred_element_type=jnp.float32)


def _scatter_sum_t(tgt_col, msgt, n):
    tile_e = msgt.shape[1]
    sselt = (tgt_col == lax.broadcasted_iota(jnp.int32, (tile_e, n), 1)
             ).astype(msgt.dtype)
    contrib = jnp.dot(msgt, sselt, preferred_element_type=jnp.float32)
    return contrib, sselt


def _seg_reduce_mat_t(chunk, width, lo, dtype):
    vv = lax.broadcasted_iota(jnp.int32, (width, chunk), 0)
    jj = lax.broadcasted_iota(jnp.int32, (width, chunk), 1)
    return ((jj // width + lo // width) == vv).astype(dtype)


def _msg0_kernel(xt_ref, fc1wt_ref, fc1bt_ref,
                 eat_ref, srcr_ref, tgtc_ref,
                 k1wt_ref, k1b_ref, k2wt_ref, k2b_ref, k3wt_ref, k3bt_ref,
                 part_ref, hout_ref, deg_ref, ewc_ref, h_vmem,
                 *, width, chunk):
    t = pl.program_id(0)
    tile_e = eat_ref.shape[1]
    n = h_vmem.shape[1]
    ww = ewc_ref.shape[0]

    @pl.when(t == 0)
    def _():
        ht = fc1wt_ref[...] * xt_ref[...] + fc1bt_ref[...]
        h_vmem[...] = ht
        hout_ref[...] = ht
        part_ref[...] = jnp.zeros(part_ref.shape, jnp.float32)
        deg_ref[...] = jnp.zeros(deg_ref.shape, jnp.float32)

    eh = jnp.maximum(jnp.dot(k1wt_ref[...], eat_ref[...],
                             preferred_element_type=jnp.float32)
                     + k1b_ref[...], 0.0)
    eh = jnp.maximum(jnp.dot(k2wt_ref[...], eh,
                             preferred_element_type=jnp.float32)
                     + k2b_ref[...], 0.0)

    hst = _gather_ht(srcr_ref[0], h_vmem[...], tile_e)
    hexpt = pltpu.repeat(hst, chunk // width, axis=0)
    msgt = jnp.zeros((width, tile_e), jnp.float32)
    for lo in range(0, ww, chunk):
        ew_c = (jnp.dot(k3wt_ref[lo:lo + chunk, :], eh,
                        preferred_element_type=jnp.float32)
                + k3bt_ref[lo:lo + chunk, :])
        ewc_ref[lo:lo + chunk, :] = ew_c.astype(jnp.bfloat16)
        prod = hexpt * ew_c
        s2t = _seg_reduce_mat_t(chunk, width, lo, jnp.float32)
        msgt = msgt + jnp.dot(s2t, prod, preferred_element_type=jnp.float32)

    contrib, sselt = _scatter_sum_t(tgtc_ref[0], msgt, n)
    part_ref[...] += contrib
    deg_ref[...] += jnp.sum(sselt, axis=0, keepdims=True)


def _msgs_kernel(hprev_ref, pprev_ref, degin_ref, roott_ref, cbt_ref,
                 ewc_ref, srcr_ref, tgtc_ref,
                 part_ref, hout_ref, h_vmem, *, width, chunk):
    t = pl.program_id(0)
    n = h_vmem.shape[1]
    tile_e = ewc_ref.shape[1]
    ww = ewc_ref.shape[0]

    @pl.when(t == 0)
    def _():
        invd = 1.0 / jnp.maximum(degin_ref[...], 1.0)
        ht = jnp.maximum(
            pprev_ref[...] * invd
            + jnp.dot(roott_ref[...], hprev_ref[...],
                      preferred_element_type=jnp.float32)
            + cbt_ref[...], 0.0)
        h_vmem[...] = ht
        hout_ref[...] = ht
        part_ref[...] = jnp.zeros(part_ref.shape, jnp.float32)

    hst = _gather_ht(srcr_ref[0], h_vmem[...], tile_e)
    hexpt = pltpu.repeat(hst.astype(jnp.bfloat16), chunk // width, axis=0)
    msgt = jnp.zeros((width, tile_e), jnp.float32)
    for lo in range(0, ww, chunk):
        prod = hexpt * ewc_ref[lo:lo + chunk, :]
        s2t = _seg_reduce_mat_t(chunk, width, lo, jnp.bfloat16)
        msgt = msgt + jnp.dot(s2t, prod, preferred_element_type=jnp.float32)

    contrib, _ = _scatter_sum_t(tgtc_ref[0], msgt, n)
    part_ref[...] += contrib


def _final_kernel(hprev_ref, pprev_ref, degin_ref, roott_ref, cbt_ref,
                  fc2wt_ref, fc2b_ref, out_ref):
    invd = 1.0 / jnp.maximum(degin_ref[...], 1.0)
    ht = jnp.maximum(
        pprev_ref[...] * invd
        + jnp.dot(roott_ref[...], hprev_ref[...],
                  preferred_element_type=jnp.float32)
        + cbt_ref[...], 0.0)
    out_ref[...] = (jnp.dot(fc2wt_ref[...], ht,
                            preferred_element_type=jnp.float32) + fc2b_ref[...])


def kernel(x, edge_index, edge_attr, fc1_w, fc1_b, k1_w, k1_b, k2_w, k2_b,
           k3_w, k3_b, root, conv_bias, fc2_w, fc2_b):
    n, _ = x.shape
    e_real, ker_in = edge_attr.shape
    width = root.shape[0]
    ww = width * width
    out_w = fc2_w.shape[1]
    tile = _TILE
    chunk = min(_CHUNK, ww)

    e_pad = ((e_real + tile - 1) // tile) * tile
    nt = e_pad // tile

    src = edge_index[0]
    tgt = edge_index[1]

    ker_in_pad = ((ker_in + 7) // 8) * 8
    eat = jnp.pad(edge_attr.astype(jnp.float32).T,
                  ((0, ker_in_pad - ker_in), (0, e_pad - e_real)))
    k1wt = jnp.pad(k1_w.T, ((0, 0), (0, ker_in_pad - ker_in)))
    srcr = jnp.pad(src, (0, e_pad - e_real)).reshape(nt, 1, tile)
    tgtc = jnp.pad(tgt, (0, e_pad - e_real),
                   constant_values=n).reshape(nt, tile, 1)

    k3wt = k3_w.reshape(-1, width, width).transpose(2, 1, 0).reshape(ww, -1)
    k3bt = k3_b.reshape(width, width).T.reshape(ww, 1)

    grid = (nt,)
    c2 = lambda t: (0, 0)
    ea_spec = pl.BlockSpec((ker_in_pad, tile), lambda t: (0, t))
    ew_spec = pl.BlockSpec((ww, tile), lambda t: (0, t))
    src_spec = pl.BlockSpec((1, 1, tile), lambda t: (t, 0, 0))
    tgt_spec = pl.BlockSpec((1, tile, 1), lambda t: (t, 0, 0))
    wn_spec = pl.BlockSpec((width, n), c2)
    dn_spec = pl.BlockSpec((1, n), c2)
    cparams = pltpu.CompilerParams(
        dimension_semantics=("arbitrary",),
        vmem_limit_bytes=_VMEM_LIMIT)
    scratch = [pltpu.VMEM((width, n), jnp.float32)]

    p, hcur, deg, ewc = pl.pallas_call(
        functools.partial(_msg0_kernel, width=width, chunk=chunk),
        grid=grid,
        in_specs=[pl.BlockSpec((1, n), c2),
                  pl.BlockSpec(fc1_w.T.shape, c2),
                  pl.BlockSpec(fc1_b.T.shape, c2),
                  ea_spec, src_spec, tgt_spec,
                  pl.BlockSpec(k1wt.shape, c2),
                  pl.BlockSpec(k1_b.T.shape, c2),
                  pl.BlockSpec(k2_w.T.shape, c2),
                  pl.BlockSpec(k2_b.T.shape, c2),
                  pl.BlockSpec(k3wt.shape, c2),
                  pl.BlockSpec(k3bt.shape, c2)],
        out_specs=[wn_spec, wn_spec, dn_spec, ew_spec],
        out_shape=[jax.ShapeDtypeStruct((width, n), jnp.float32),
                   jax.ShapeDtypeStruct((width, n), jnp.float32),
                   jax.ShapeDtypeStruct((1, n), jnp.float32),
                   jax.ShapeDtypeStruct((ww, e_pad), jnp.bfloat16)],
        scratch_shapes=scratch,
        compiler_params=cparams,
    )(x.astype(jnp.float32).T, fc1_w.T, fc1_b.T, eat, srcr, tgtc,
      k1wt, k1_b.T, k2_w.T, k2_b.T, k3wt, k3bt)

    for _ in range(_DEPTH - 1):
        p, hcur = pl.pallas_call(
            functools.partial(_msgs_kernel, width=width, chunk=chunk),
            grid=grid,
            in_specs=[wn_spec, wn_spec, dn_spec,
                      pl.BlockSpec(root.shape, c2),
                      pl.BlockSpec(conv_bias.T.shape, c2),
                      ew_spec, src_spec, tgt_spec],
            out_specs=[wn_spec, wn_spec],
            out_shape=[jax.ShapeDtypeStruct((width, n), jnp.float32),
                       jax.ShapeDtypeStruct((width, n), jnp.float32)],
            scratch_shapes=scratch,
            compiler_params=cparams,
        )(hcur, p, deg, root.T, conv_bias.T, ewc, srcr, tgtc)

    outt = pl.pallas_call(
        _final_kernel,
        out_shape=jax.ShapeDtypeStruct((out_w, n), jnp.float32),
    )(hcur, p, deg, root.T, conv_bias.T, fc2_w.T, fc2_b.T)
    return outt.T
